# baseline (speedup 1.0000x reference)
import functools

import jax
import jax.numpy as jnp
from jax import lax
from jax.experimental import pallas as pl
from jax.experimental.pallas import tpu as pltpu
from jax.experimental.pallas import tpu_sc as plsc

_NW = 32
_L = 16
_EPV = 120


def _project_body(nra, m_ref, bn_ref, bc_ref, bias_ref, p_ref, c_ref):
    m = m_ref[...]
    res = jnp.dot(m, bn_ref[...], preferred_element_type=jnp.float32)
    for ra in range(nra):
        p_ref[ra] = res[:, ra * 32:(ra + 1) * 32]
    c_ref[...] = (
        jnp.dot(m, bc_ref[...], preferred_element_type=jnp.float32)
        + bias_ref[...]
    )


def _project(mesh_pad, bn, bc, bias_row, np_, nra, blk_m):
    grid = (pl.cdiv(np_, blk_m),)
    return pl.pallas_call(
        functools.partial(_project_body, nra),
        grid=grid,
        in_specs=[
            pl.BlockSpec((blk_m, mesh_pad.shape[1]), lambda i: (i, 0)),
            pl.BlockSpec(bn.shape, lambda i: (0, 0)),
            pl.BlockSpec(bc.shape, lambda i: (0, 0)),
            pl.BlockSpec((1, bias_row.shape[1]), lambda i: (0, 0)),
        ],
        out_specs=[
            pl.BlockSpec((nra, blk_m, 32), lambda i: (0, i, 0)),
            pl.BlockSpec((blk_m, bc.shape[1]), lambda i: (i, 0)),
        ],
        out_shape=[
            jax.ShapeDtypeStruct((nra, np_, 32), jnp.float32),
            jax.ShapeDtypeStruct((np_, bc.shape[1]), jnp.float32),
        ],
    )(mesh_pad, bn, bc, bias_row)


def _make_sc_kernel(nv_t, nra):
    ne_t = nv_t * _EPV
    nvec = ne_t // _L
    mesh = plsc.VectorSubcoreMesh(
        core_axis_name="c", subcore_axis_name="s",
        num_cores=2, num_subcores=16)

    @functools.partial(
        pl.kernel,
        out_type=jax.ShapeDtypeStruct((nv_t * _NW * 32,), jnp.float32),
        mesh=mesh,
        compiler_params=pltpu.CompilerParams(
            needs_layout_passes=False, use_tc_tiling_on_sc=False),
        scratch_types=[
            pltpu.VMEM((ne_t,), jnp.int32),
            pltpu.VMEM((ne_t,), jnp.float32),
            pltpu.VMEM((2 * _EPV,), jnp.int32),
            pltpu.VMEM((_EPV, 32), jnp.float32),
            pltpu.VMEM((_EPV, 32), jnp.float32),
            pltpu.VMEM((nv_t * 32,), jnp.float32),
            pltpu.SemaphoreType.DMA,
            pltpu.SemaphoreType.DMA,
        ],
    )
    def sc_kernel(tab, idxh, wh, rah, ch, out,
                  idxb, wb, rab, g_a, g_b, outb, sem_a, sem_b):
        wid = lax.axis_index("s") * 2 + lax.axis_index("c")
        v0 = wid * nv_t
        e0 = v0 * _EPV

        pltpu.sync_copy(idxh.at[pl.ds(e0, ne_t)], idxb)
        pltpu.sync_copy(wh.at[pl.ds(e0, ne_t)], wb)
        pltpu.sync_copy(rah, rab)
        pltpu.sync_copy(ch.at[pl.ds(v0 * 32, nv_t * 32)], outb)

        def flat_body(i, _):
            q = lax.rem(i, 15)
            v = idxb[pl.ds(i * _L, _L)]
            r = rab[pl.ds(q * _L, _L)]
            idxb[pl.ds(i * _L, _L)] = v + r
            return 0

        lax.fori_loop(0, nvec, flat_body, 0, unroll=4)

        def fire(vl, gbuf, sem):
            idx_slice = idxb.at[pl.ds(vl * _EPV, _EPV)]
            return pltpu.async_copy(tab.at[idx_slice], gbuf, sem)

        def wait(vl, gbuf, sem):
            idx_slice = idxb.at[pl.ds(vl * _EPV, _EPV)]
            pltpu.make_async_copy(tab.at[idx_slice], gbuf, sem).wait()

        def accumulate(vl, gbuf):
            base_e = vl * _EPV
            base_o = vl * 32
            acc0_i = outb[pl.ds(base_o, _L)]
            acc1_i = outb[pl.ds(base_o + _L, _L)]

            def acc_body(j, carry):
                a0, a1 = carry
                for u in range(8):
                    e = j * 8 + u
                    wv = plsc.load_gather(
                        wb, [lax.broadcast(base_e + e, (_L,))])
                    r0 = gbuf[e, pl.ds(0, _L)]
                    r1 = gbuf[e, pl.ds(_L, _L)]
                    a0 = a0 + wv * r0
                    a1 = a1 + wv * r1
                return (a0, a1)

            a0, a1 = lax.fori_loop(0, _EPV // 8, acc_body, (acc0_i, acc1_i))
            zero = jnp.zeros((_L,), jnp.float32)
            outb[pl.ds(base_o, _L)] = jnp.maximum(a0, zero)
            outb[pl.ds(base_o + _L, _L)] = jnp.maximum(a1, zero)

        fire(0, g_a, sem_a)

        def pair_body(v2, _):
            vl = v2 * 2
            fire(vl + 1, g_b, sem_b)
            wait(vl, g_a, sem_a)
            accumulate(vl, g_a)

            @pl.when(v2 < nv_t // 2 - 1)
            def _():
                fire(vl + 2, g_a, sem_a)

            wait(vl + 1, g_b, sem_b)
            accumulate(vl + 1, g_b)
            return 0

        lax.fori_loop(0, nv_t // 2, pair_body, 0)

        pltpu.sync_copy(outb, out.at[pl.ds(v0 * 32, nv_t * 32)])

    return sc_kernel


def _prep(mesh_signal, bary_coordinates, neighbor_weights, self_weights,
          bias):
    n, f = mesh_signal.shape
    t, r, a, _ = neighbor_weights.shape
    nj = a // 2
    nra = r * a
    epv = nra * 3
    assert epv == _EPV and nj * t == 32

    nv_t = 2 * ((n + 2 * _NW - 1) // (2 * _NW))
    np_ = nv_t * _NW

    wrot = jnp.stack(
        [jnp.roll(neighbor_weights, -2 * j, axis=2) for j in range(nj)],
        axis=0)
    bn = wrot.transpose(2, 3, 0, 1, 4).reshape(nra * nj * t, f).T
    bc = jnp.tile(self_weights[:, 0, :], (nj, 1)).T
    bias_row = jnp.tile(bias, (nj,)).reshape(1, nj * t)

    mesh_pad = jnp.pad(mesh_signal, ((0, np_ - n), (0, 0)))
    idx_i = bary_coordinates[..., 0].astype(jnp.int32).reshape(n, epv)
    w_f = bary_coordinates[..., 1].reshape(n, epv)
    idx_i = jnp.pad(idx_i, ((0, np_ - n), (0, 0))).reshape(np_ * epv)
    w_f = jnp.pad(w_f, ((0, np_ - n), (0, 0))).reshape(np_ * epv)
    ra_pat = jnp.tile(jnp.repeat(jnp.arange(nra, dtype=jnp.int32), 3), 2) * np_
    return (mesh_pad, bn, bc, bias_row, idx_i, w_f, ra_pat,
            n, nj, t, nra, nv_t, np_)


def kernel(mesh_signal, bary_coordinates, neighbor_weights, self_weights,
           bias):
    (mesh_pad, bn, bc, bias_row, idx_i, w_f, ra_pat,
     n, nj, t, nra, nv_t, np_) = _prep(
        mesh_signal, bary_coordinates, neighbor_weights, self_weights, bias)

    p3, c = _project(mesh_pad, bn, bc, bias_row, np_, nra, np_ // 8)
    tab = p3.reshape(nra * np_, nj * t)

    sck = _make_sc_kernel(nv_t, nra)
    out_flat = sck(tab, idx_i, w_f, ra_pat, c.reshape(np_ * nj * t))

    return out_flat.reshape(np_, nj, t)[:n]


# speedup vs baseline: 1.3435x; 1.3435x over previous
import functools

import jax
import jax.numpy as jnp
from jax import lax
from jax.experimental import pallas as pl
from jax.experimental.pallas import tpu as pltpu
from jax.experimental.pallas import tpu_sc as plsc

_NW = 32
_L = 16
_EPV = 121
_NBLK = 41


def _project_body(m_ref, b_ref, o_ref):
    o_ref[...] = jnp.dot(
        m_ref[...], b_ref[...], preferred_element_type=jnp.float32
    ).astype(jnp.bfloat16)


def _project(mesh_pad, bperm, np_, blk_m):
    nc = bperm.shape[1]
    return pl.pallas_call(
        _project_body,
        grid=(pl.cdiv(np_, blk_m),),
        in_specs=[
            pl.BlockSpec((blk_m, mesh_pad.shape[1]), lambda i: (i, 0)),
            pl.BlockSpec(bperm.shape, lambda i: (0, 0)),
        ],
        out_specs=pl.BlockSpec((blk_m, nc), lambda i: (i, 0)),
        out_shape=jax.ShapeDtypeStruct((np_, nc), jnp.bfloat16),
    )(mesh_pad, bperm)


def _make_sc_kernel(nv_t):
    nraw = nv_t * 2 * _EPV
    mesh = plsc.VectorSubcoreMesh(
        core_axis_name="c", subcore_axis_name="s",
        num_cores=2, num_subcores=16)

    @functools.partial(
        pl.kernel,
        out_type=jax.ShapeDtypeStruct((nv_t * _NW * 32,), jnp.float32),
        mesh=mesh,
        compiler_params=pltpu.CompilerParams(
            needs_layout_passes=False, use_tc_tiling_on_sc=False),
        scratch_types=[
            pltpu.VMEM((nraw + 16,), jnp.float32),
            pltpu.VMEM((128,), jnp.int32),
            pltpu.VMEM((128,), jnp.int32),
            pltpu.VMEM((_EPV, 32), jnp.bfloat16),
            pltpu.VMEM((_EPV, 32), jnp.bfloat16),
            pltpu.VMEM((128,), jnp.int32),
            pltpu.VMEM((32,), jnp.float32),
            pltpu.VMEM((nv_t * 32,), jnp.float32),
            pltpu.SemaphoreType.DMA,
            pltpu.SemaphoreType.DMA,
        ],
    )
    def sc_kernel(tab, rawh, path, biash, out,
                  rawb, f_a, f_b, g_a, g_b, patb, biasb, outb, sem_a, sem_b):
        wid = lax.axis_index("s") * 2 + lax.axis_index("c")
        v0 = wid * nv_t

        pltpu.sync_copy(rawh.at[pl.ds(v0 * 2 * _EPV, nraw)],
                        rawb.at[pl.ds(0, nraw)])
        pltpu.sync_copy(path, patb)
        pltpu.sync_copy(biash, biasb)

        bias_a = biasb[pl.ds(0, _L)]
        bias_b = biasb[pl.ds(_L, _L)]
        zero = jnp.zeros((_L,), jnp.float32)
        iota2 = lax.iota(jnp.int32, _L) * 2

        def compute_flat(vl, fbuf):
            base = vl * 2 * _EPV
            for u in range(8):
                iv = lax.broadcast(base + 32 * u, (_L,)) + iota2
                idxf = plsc.load_gather(rawb, [iv])
                fbuf[pl.ds(u * _L, _L)] = (
                    idxf.astype(jnp.int32) * _NBLK + patb[pl.ds(u * _L, _L)])

        def fire(fbuf, gbuf, sem):
            pltpu.async_copy(tab.at[fbuf.at[pl.ds(0, _EPV)]], gbuf, sem)

        def wait(fbuf, gbuf, sem):
            pltpu.make_async_copy(
                tab.at[fbuf.at[pl.ds(0, _EPV)]], gbuf, sem).wait()

        def accumulate(vl, gbuf):
            base = vl * 2 * _EPV

            def acc_body(j, carry):
                accs = list(carry)
                wb = lax.broadcast(base + 16 * j + 1, (_L,))
                for u in range(8):
                    e = j * 8 + u
                    wv = plsc.load_gather(rawb, [wb + (2 * u)])
                    a, b = plsc.unpack(
                        gbuf[e, pl.ds(0, 32)],
                        format=plsc.PackFormat.INTERLEAVED)
                    k = 2 * (u % 4)
                    accs[k] = accs[k] + wv * a
                    accs[k + 1] = accs[k + 1] + wv * b
                return tuple(accs)

            init = (bias_a, bias_b) + (zero,) * 6
            accs = lax.fori_loop(0, (_EPV - 1) // 8, acc_body, init)
            a, b = plsc.unpack(gbuf[_EPV - 1, pl.ds(0, 32)],
                               format=plsc.PackFormat.INTERLEAVED)
            acc_a = (accs[0] + a) + (accs[2] + accs[4]) + accs[6]
            acc_b = (accs[1] + b) + (accs[3] + accs[5]) + accs[7]
            outb[pl.ds(vl * 32, _L)] = jnp.maximum(acc_a, zero)
            outb[pl.ds(vl * 32 + _L, _L)] = jnp.maximum(acc_b, zero)

        compute_flat(0, f_a)
        fire(f_a, g_a, sem_a)

        def pair_body(v2, _):
            vl = v2 * 2
            compute_flat(vl + 1, f_b)
            fire(f_b, g_b, sem_b)
            wait(f_a, g_a, sem_a)
            accumulate(vl, g_a)

            @pl.when(v2 < nv_t // 2 - 1)
            def _():
                compute_flat(vl + 2, f_a)
                fire(f_a, g_a, sem_a)

            wait(f_b, g_b, sem_b)
            accumulate(vl + 1, g_b)
            return 0

        lax.fori_loop(0, nv_t // 2, pair_body, 0)

        pltpu.sync_copy(outb, out.at[pl.ds(v0 * 32, nv_t * 32)])

    return sc_kernel


def _prep(mesh_signal, bary_coordinates, neighbor_weights, self_weights,
          bias):
    n, f = mesh_signal.shape
    t, r, a, _ = neighbor_weights.shape
    nj = a // 2
    nra = r * a
    assert nra * 3 + 1 == _EPV and nj * t == 32 and nra + 1 == _NBLK

    nv_t = 4 * ((n + 4 * _NW - 1) // (4 * _NW))
    np_ = nv_t * _NW

    wrot = jnp.stack(
        [jnp.roll(neighbor_weights, -2 * j, axis=2) for j in range(nj)],
        axis=0)
    bn = wrot.transpose(4, 2, 3, 0, 1).reshape(f, nra, nj * t)
    bc = jnp.tile(self_weights[:, 0, :], (nj, 1)).T[:, None, :]
    blog = jnp.concatenate([bn, bc], axis=1)
    colperm = jnp.stack(
        [jnp.arange(16), jnp.arange(16) + 16], axis=1).reshape(32)
    bperm = blog[:, :, colperm].reshape(f, _NBLK * 32)
    bias32 = jnp.tile(bias, (nj,))

    bary2 = jnp.pad(bary_coordinates.reshape(n, nra * 6),
                    ((0, np_ - n), (0, 0)))
    selfcol = jnp.stack(
        [jnp.arange(np_, dtype=jnp.float32),
         jnp.ones((np_,), jnp.float32)], axis=1)
    raw = jnp.concatenate([bary2, selfcol], axis=1).reshape(np_ * 2 * _EPV)

    pat = jnp.concatenate([
        jnp.repeat(jnp.arange(nra, dtype=jnp.int32), 3),
        jnp.full((8,), nra, dtype=jnp.int32)])

    mesh_pad = jnp.pad(mesh_signal, ((0, np_ - n), (0, 0)))
    return mesh_pad, bperm, bias32, raw, pat, n, nj, t, nv_t, np_


def kernel(mesh_signal, bary_coordinates, neighbor_weights, self_weights,
           bias):
    (mesh_pad, bperm, bias32, raw, pat, n, nj, t, nv_t, np_) = _prep(
        mesh_signal, bary_coordinates, neighbor_weights, self_weights, bias)

    p = _project(mesh_pad, bperm, np_, np_ // 8)
    tab = p.reshape(np_ * _NBLK, nj * t)

    sck = _make_sc_kernel(nv_t)
    out_flat = sck(tab, raw, pat, bias32)

    return out_flat.reshape(np_, nj, t)[:n]
